# baseline (speedup 1.0000x reference)
import jax
import jax.numpy as jnp
from jax.experimental import pallas as pl


def _copy_kernel(x_ref, o_ref):
    o_ref[...] = x_ref[...]


def _lin(x, W, b):
    return x @ W + b


def _bn(x, g, b):
    return (x - x.mean(0)) / jnp.sqrt(x.var(0) + 1e-5) * g + b


def kernel(h, edge_weight, graph_x, params, edge_index, graph_edge_index, graph_batch):
    p = params
    G = 10000
    D = 128
    N2 = graph_x.shape[0]
    src, dst = graph_edge_index[0], graph_edge_index[1]
    vnode = jnp.zeros((G, D), jnp.float32)
    x = graph_x
    for l in range(5):
        x = x + vnode[graph_batch]
        msg = jnp.zeros_like(x).at[dst].add(jax.nn.relu(x[src]))
        z = _lin(x + msg, p['gin_W1'][l], p['gin_b1'][l])
        z = jax.nn.relu(_bn(z, p['gin_bn1_g'][l], p['gin_bn1_b'][l]))
        z = _lin(z, p['gin_W2'][l], p['gin_b2'][l])
        z = _bn(z, p['gin_bng'][l], p['gin_bnb'][l])
        if l < 4:
            z = jax.nn.relu(z)
        x = z + x
        if l < 4:
            pooled = jax.ops.segment_sum(x, graph_batch, num_segments=G) + vnode
            t = jax.nn.relu(_lin(pooled, p['vn_W1'][l], p['vn_b1'][l]))
            vnode = vnode + jax.nn.relu(_lin(t, p['vn_W2'][l], p['vn_b2'][l]))
    cnt = jax.ops.segment_sum(jnp.ones((N2,), jnp.float32), graph_batch, num_segments=G)
    pre_h = jax.ops.segment_sum(x, graph_batch, num_segments=G) / jnp.clip(cnt, 1.0)[:, None]
    s2, d2 = edge_index[0], edge_index[1]
    hh = h
    for i in range(2):
        msg = jnp.zeros_like(hh).at[d2].add(edge_weight[i][:, None] * hh[s2])
        z = _lin(hh + msg, p['s_W1'][i], p['s_b1'][i])
        z = jax.nn.relu(_bn(z, p['s_bn1_g'][i], p['s_bn1_b'][i]))
        z = _lin(z, p['s_W2'][i], p['s_b2'][i])
        z = jax.nn.relu(_bn(z, p['s_bnA_g'][i], p['s_bnA_b'][i]))
        z = jax.nn.relu(_bn(z, p['s_bno_g'][i], p['s_bno_b'][i]))
        hh = z
    y = jnp.concatenate([hh, pre_h], axis=1)
    y = pl.pallas_call(
        _copy_kernel,
        out_shape=jax.ShapeDtypeStruct(y.shape, y.dtype),
    )(y)
    y = jax.nn.relu(_lin(y, p['mr_W0'], p['mr_b0']))
    y = jax.nn.relu(_lin(y, p['mr_W1'], p['mr_b1']))
    y = _lin(y, p['mr_W2'], p['mr_b2'])
    return y


# speedup vs baseline: 1.5912x; 1.5912x over previous
import functools

import jax
import jax.numpy as jnp
from jax import lax
from jax.experimental import pallas as pl
from jax.experimental.pallas import tpu as pltpu
from jax.experimental.pallas import tpu_sc as plsc

NW = 32
CH = 128
D = 128
NG = 10000
ACC = 10112
ZR = 632


def _mesh():
    return plsc.VectorSubcoreMesh(core_axis_name="c", subcore_axis_name="s",
                                  num_cores=2)


def _extract(g0, g1, i):
    io = lax.broadcasted_iota(jnp.int32, (16,), 0)
    z = jnp.zeros((16,), jnp.int32)
    a = jnp.sum(jnp.where(io == i, g0, z), axis=0)
    b = jnp.sum(jnp.where(io == (i - 16), g1, z), axis=0)
    return a + b


def _sc_gather_add(table, x, idx):
    n_rows = x.shape[0]
    nch = n_rows // CH
    kmax = (nch + NW - 1) // NW

    @functools.partial(
        pl.kernel,
        mesh=_mesh(),
        out_type=[jax.ShapeDtypeStruct((n_rows, D), jnp.float32),
                  jax.ShapeDtypeStruct((n_rows, D), jnp.float32)],
        scratch_types=[
            pltpu.VMEM((CH,), jnp.int32),
            pltpu.VMEM((CH, D), jnp.float32),
            pltpu.VMEM((CH, D), jnp.float32),
            pltpu.VMEM((CH, D), jnp.float32),
            pltpu.SemaphoreType.DMA,
        ],
    )
    def k(tab_h, x_h, idx_h, xu_h, xur_h, idx_v, gbuf, xbuf, rbuf, sem):
        tid = lax.axis_index("s") * 2 + lax.axis_index("c")

        def body(kk, car):
            c = tid + kk * NW

            @pl.when(c < nch)
            def _do():
                base = c * CH
                pltpu.sync_copy(idx_h.at[pl.ds(base, CH)], idx_v)
                cp = pltpu.async_copy(tab_h.at[idx_v], gbuf, sem)
                pltpu.sync_copy(x_h.at[pl.ds(base, CH)], xbuf)
                cp.wait()

                def rloop(r, c2):
                    for j in range(D // 16):
                        sl = pl.ds(j * 16, 16)
                        v = xbuf[r, sl] + gbuf[r, sl]
                        xbuf[r, sl] = v
                        rbuf[r, sl] = jnp.maximum(v, 0.0)
                    return c2

                lax.fori_loop(0, CH, rloop, 0)
                pltpu.sync_copy(xbuf, xu_h.at[pl.ds(base, CH)])
                pltpu.sync_copy(rbuf, xur_h.at[pl.ds(base, CH)])
            return car

        lax.fori_loop(0, kmax, body, 0)

    return k(table, x, idx)


def _sc_msg_sorted(xur, src_s, dst_s, ptr, zeros, n_nodes):
    e = src_s.shape[0]
    nwin = n_nodes // NG

    @functools.partial(
        pl.kernel,
        mesh=_mesh(),
        out_type=jax.ShapeDtypeStruct((n_nodes, D), jnp.float32),
        scratch_types=[
            pltpu.VMEM((32,), jnp.int32),
            pltpu.VMEM((CH,), jnp.int32),
            pltpu.VMEM((CH,), jnp.int32),
            pltpu.VMEM((CH,), jnp.int32),
            pltpu.VMEM((CH, D), jnp.float32),
            pltpu.VMEM_SHARED((ACC, D), jnp.float32),
            pltpu.SemaphoreType.DMA,
        ],
    )
    def k(tab_h, src_h, dst_h, ptr_h, z_h, out_h,
          ptr_v, idx_v, dst_v, sidx, gbuf, acc, gsem):
        cid = lax.axis_index("c")
        sid = lax.axis_index("s")
        pltpu.sync_copy(ptr_h, ptr_v)
        io = lax.broadcasted_iota(jnp.int32, (16,), 0)

        for wi in range(nwin // 2):
            w = 2 * wi + cid
            lo = w * NG
            pv = ptr_v[pl.ds(w, 16)]
            e0 = pv[0]
            e1 = pv[1]
            pltpu.sync_copy(z_h, acc.at[pl.ds(sid * ZR, ZR)])
            plsc.subcore_barrier()
            per = (e1 - e0 + 15) // 16
            m0 = e0 + sid * per
            m1 = jnp.minimum(m0 + per, e1)
            k0 = m0 // CH
            k1 = (m1 + CH - 1) // CH

            def cloop(kc, car):
                a = kc * CH
                pltpu.sync_copy(src_h.at[pl.ds(a, CH)], idx_v)
                cp = pltpu.async_copy(tab_h.at[idx_v], gbuf, gsem)
                pltpu.sync_copy(dst_h.at[pl.ds(a, CH)], dst_v)
                for j in range(CH // 16):
                    sl = pl.ds(j * 16, 16)
                    eix = io + (a + j * 16)
                    valid = (eix >= m0) & (eix < m1)
                    sidx[sl] = jnp.where(valid, dst_v[sl] - lo,
                                         ACC - 16 + (eix & 7))
                cp.wait()
                pltpu.sync_copy(gbuf, acc.at[sidx], add=True)
                return car

            lax.fori_loop(k0, k1, cloop, 0)
            plsc.subcore_barrier()
            pltpu.sync_copy(acc.at[pl.ds(sid * 624, 624)],
                            out_h.at[pl.ds(lo + sid * 624, 624)])

            @pl.when(sid == 15)
            def _tail():
                pltpu.sync_copy(acc.at[pl.ds(9984, 16)],
                                out_h.at[pl.ds(lo + 9984, 16)])

            plsc.subcore_barrier()

    return k(xur, src_s, dst_s, ptr, zeros)


def _sc_msg_weighted(hh, src_p, dst_p, ew_p, zeros, n_edges_real):
    e_pad = src_p.shape[0]
    per_tile = n_edges_real // NW
    nchk = (per_tile + CH - 1) // CH

    @functools.partial(
        pl.kernel,
        mesh=_mesh(),
        out_type=jax.ShapeDtypeStruct((2, ACC, D), jnp.float32),
        scratch_types=[
            pltpu.VMEM((CH,), jnp.int32),
            pltpu.VMEM((CH,), jnp.int32),
            pltpu.VMEM((CH,), jnp.int32),
            pltpu.VMEM((CH,), jnp.float32),
            pltpu.VMEM((CH, D), jnp.float32),
            pltpu.VMEM_SHARED((ACC, D), jnp.float32),
            pltpu.SemaphoreType.DMA,
        ],
    )
    def k(tab_h, src_h, dst_h, ew_h, z_h, out_h,
          idx_v, dst_v, sidx, wbuf, gbuf, acc, gsem):
        cid = lax.axis_index("c")
        sid = lax.axis_index("s")
        tid = cid * 16 + sid
        base = tid * per_tile
        m1 = base + per_tile
        pltpu.sync_copy(z_h, acc.at[pl.ds(sid * ZR, ZR)])
        plsc.subcore_barrier()
        io = lax.broadcasted_iota(jnp.int32, (16,), 0)

        def cloop(kc, car):
            a = base + kc * CH
            pltpu.sync_copy(src_h.at[pl.ds(a, CH)], idx_v)
            cp = pltpu.async_copy(tab_h.at[idx_v], gbuf, gsem)
            pltpu.sync_copy(dst_h.at[pl.ds(a, CH)], dst_v)
            pltpu.sync_copy(ew_h.at[pl.ds(a, CH)], wbuf)
            for j in range(CH // 16):
                sl = pl.ds(j * 16, 16)
                eix = io + (a + j * 16)
                valid = eix < m1
                sidx[sl] = jnp.where(valid, dst_v[sl], ACC - 16 + (eix & 7))
            cp.wait()

            def gloop(g, car2):
                wg = wbuf[pl.ds(g * 16, 16)]
                for kk in range(16):
                    r = g * 16 + kk
                    wb = lax.gather(
                        wg,
                        jnp.full((16, 1), kk, jnp.int32),
                        lax.GatherDimensionNumbers(
                            offset_dims=(), collapsed_slice_dims=(0,),
                            start_index_map=(0,)),
                        (1,), mode=lax.GatherScatterMode.PROMISE_IN_BOUNDS)
                    for j in range(D // 16):
                        sl = pl.ds(j * 16, 16)
                        gbuf[r, sl] = gbuf[r, sl] * wb
                return car2

            lax.fori_loop(0, CH // 16, gloop, 0)
            pltpu.sync_copy(gbuf, acc.at[sidx], add=True)
            return car

        lax.fori_loop(0, nchk, cloop, 0)
        plsc.subcore_barrier()
        pltpu.sync_copy(acc.at[pl.ds(sid * ZR, ZR)],
                        out_h.at[cid, pl.ds(sid * ZR, ZR)])

    return k(hh, src_p, dst_p, ew_p, zeros)


def _sc_pool(x, gb, zeros):
    n_rows = x.shape[0]
    nch_half = (n_rows // 2) // CH

    @functools.partial(
        pl.kernel,
        mesh=_mesh(),
        out_type=jax.ShapeDtypeStruct((2, ACC, D), jnp.float32),
        scratch_types=[
            pltpu.VMEM((CH,), jnp.int32),
            pltpu.VMEM((CH, D), jnp.float32),
            pltpu.VMEM_SHARED((ACC, D), jnp.float32),
        ],
    )
    def k(x_h, gb_h, z_h, out_h, gb_v, gbuf, acc):
        cid = lax.axis_index("c")
        sid = lax.axis_index("s")
        pltpu.sync_copy(z_h, acc.at[pl.ds(sid * ZR, ZR)])
        plsc.subcore_barrier()
        kmax = (nch_half + 15) // 16

        def cloop(kk, car):
            c = sid + kk * 16

            @pl.when(c < nch_half)
            def _do():
                base = (cid * nch_half + c) * CH
                pltpu.sync_copy(x_h.at[pl.ds(base, CH)], gbuf)
                pltpu.sync_copy(gb_h.at[pl.ds(base, CH)], gb_v)
                pltpu.sync_copy(gbuf, acc.at[gb_v], add=True)
            return car

        lax.fori_loop(0, kmax, cloop, 0)
        plsc.subcore_barrier()
        pltpu.sync_copy(acc.at[pl.ds(sid * ZR, ZR)],
                        out_h.at[cid, pl.ds(sid * ZR, ZR)])

    return k(x, gb, zeros)


def _mm_stats(x, adds, W, b, scale, shift, *, relu_pre, br):
    rows, kdim = x.shape
    ncols = W.shape[1]
    grid = (rows // br,)
    n_add = len(adds)
    has_aff = scale is not None

    def body(*refs):
        it = iter(refs)
        x_ref = next(it)
        add_refs = [next(it) for _ in range(n_add)]
        sc_ref = next(it) if has_aff else None
        sh_ref = next(it) if has_aff else None
        w_ref = next(it)
        b_ref = next(it)
        y_ref = next(it)
        st_ref = next(it)
        xx = x_ref[...]
        for ar in add_refs:
            xx = xx + ar[...]
        if has_aff:
            xx = xx * sc_ref[...] + sh_ref[...]
            if relu_pre:
                xx = jnp.maximum(xx, 0.0)
        y = jnp.dot(xx, w_ref[...], preferred_element_type=jnp.float32)
        y = y + b_ref[...]
        y_ref[...] = y

        @pl.when(pl.program_id(0) == 0)
        def _init():
            st_ref[...] = jnp.zeros_like(st_ref)

        st_ref[0:1, :] += jnp.sum(y, axis=0, keepdims=True)
        st_ref[1:2, :] += jnp.sum(y * y, axis=0, keepdims=True)

    in_specs = [pl.BlockSpec((br, kdim), lambda i: (i, 0))]
    ins = [x]
    for a in adds:
        in_specs.append(pl.BlockSpec((br, kdim), lambda i: (i, 0)))
        ins.append(a)
    if has_aff:
        in_specs += [pl.BlockSpec((1, kdim), lambda i: (0, 0))] * 2
        ins += [scale.reshape(1, kdim), shift.reshape(1, kdim)]
    in_specs += [pl.BlockSpec((kdim, ncols), lambda i: (0, 0)),
                 pl.BlockSpec((1, ncols), lambda i: (0, 0))]
    ins += [W, b.reshape(1, ncols)]

    y, st = pl.pallas_call(
        body,
        grid=grid,
        in_specs=in_specs,
        out_specs=[pl.BlockSpec((br, ncols), lambda i: (i, 0)),
                   pl.BlockSpec((8, ncols), lambda i: (0, 0))],
        out_shape=[jax.ShapeDtypeStruct((rows, ncols), jnp.float32),
                   jax.ShapeDtypeStruct((8, ncols), jnp.float32)],
    )(*ins)
    return y, st


def _affine_ew(z, scale, shift, res, *, relu, stats, br):
    rows, ncols = z.shape
    grid = (rows // br,)
    has_res = res is not None

    def body(*refs):
        it = iter(refs)
        z_ref = next(it)
        sc_ref = next(it)
        sh_ref = next(it)
        r_ref = next(it) if has_res else None
        y_ref = next(it)
        st_ref = next(it) if stats else None
        y = z_ref[...] * sc_ref[...] + sh_ref[...]
        if relu:
            y = jnp.maximum(y, 0.0)
        if has_res:
            y = y + r_ref[...]
        y_ref[...] = y
        if stats:
            @pl.when(pl.program_id(0) == 0)
            def _init():
                st_ref[...] = jnp.zeros_like(st_ref)
            st_ref[0:1, :] += jnp.sum(y, axis=0, keepdims=True)
            st_ref[1:2, :] += jnp.sum(y * y, axis=0, keepdims=True)

    in_specs = [pl.BlockSpec((br, ncols), lambda i: (i, 0)),
                pl.BlockSpec((1, ncols), lambda i: (0, 0)),
                pl.BlockSpec((1, ncols), lambda i: (0, 0))]
    ins = [z, scale.reshape(1, ncols), shift.reshape(1, ncols)]
    if has_res:
        in_specs.append(pl.BlockSpec((br, ncols), lambda i: (i, 0)))
        ins.append(res)
    out_specs = [pl.BlockSpec((br, ncols), lambda i: (i, 0))]
    out_shape = [jax.ShapeDtypeStruct((rows, ncols), jnp.float32)]
    if stats:
        out_specs.append(pl.BlockSpec((8, ncols), lambda i: (0, 0)))
        out_shape.append(jax.ShapeDtypeStruct((8, ncols), jnp.float32))
    out = pl.pallas_call(
        body, grid=grid, in_specs=in_specs, out_specs=out_specs,
        out_shape=out_shape)(*ins)
    return out if stats else out[0]


def _vnode_mlp(p0, p1, vn, W1, b1, W2, b2, *, br):
    rows = vn.shape[0]
    grid = (rows // br,)

    def body(p0_ref, p1_ref, v_ref, w1_ref, b1_ref, w2_ref, b2_ref, o_ref):
        t = jnp.dot(p0_ref[...] + p1_ref[...] + v_ref[...], w1_ref[...],
                    preferred_element_type=jnp.float32) + b1_ref[...]
        t = jnp.maximum(t, 0.0)
        u = jnp.dot(t, w2_ref[...],
                    preferred_element_type=jnp.float32) + b2_ref[...]
        o_ref[...] = v_ref[...] + jnp.maximum(u, 0.0)

    return pl.pallas_call(
        body, grid=grid,
        in_specs=[pl.BlockSpec((br, D), lambda i: (i, 0)),
                  pl.BlockSpec((br, D), lambda i: (i, 0)),
                  pl.BlockSpec((br, D), lambda i: (i, 0)),
                  pl.BlockSpec((D, 2 * D), lambda i: (0, 0)),
                  pl.BlockSpec((1, 2 * D), lambda i: (0, 0)),
                  pl.BlockSpec((2 * D, D), lambda i: (0, 0)),
                  pl.BlockSpec((1, D), lambda i: (0, 0))],
        out_specs=pl.BlockSpec((br, D), lambda i: (i, 0)),
        out_shape=jax.ShapeDtypeStruct((rows, D), jnp.float32),
    )(p0, p1, vn, W1, b1.reshape(1, -1), W2, b2.reshape(1, -1))


def _readout(hh, p0, p1, cinv, W0a, W0b, b0, W1, b1, W2, b2, *, br):
    rows = hh.shape[0]
    grid = (rows // br,)

    def body(h_ref, p0_ref, p1_ref, c_ref, wa_ref, wb_ref, b0_ref,
             w1_ref, b1_ref, w2_ref, b2_ref, o_ref):
        ph = (p0_ref[...] + p1_ref[...]) * c_ref[...]
        y = (jnp.dot(h_ref[...], wa_ref[...], preferred_element_type=jnp.float32)
             + jnp.dot(ph, wb_ref[...], preferred_element_type=jnp.float32)
             + b0_ref[...])
        y = jnp.maximum(y, 0.0)
        y = jnp.dot(y, w1_ref[...], preferred_element_type=jnp.float32) + b1_ref[...]
        y = jnp.maximum(y, 0.0)
        y = jnp.dot(y, w2_ref[...], preferred_element_type=jnp.float32) + b2_ref[...]
        o_ref[...] = y

    return pl.pallas_call(
        body, grid=grid,
        in_specs=[pl.BlockSpec((br, D), lambda i: (i, 0)),
                  pl.BlockSpec((br, D), lambda i: (i, 0)),
                  pl.BlockSpec((br, D), lambda i: (i, 0)),
                  pl.BlockSpec((br, 1), lambda i: (i, 0)),
                  pl.BlockSpec((D, D), lambda i: (0, 0)),
                  pl.BlockSpec((D, D), lambda i: (0, 0)),
                  pl.BlockSpec((1, D), lambda i: (0, 0)),
                  pl.BlockSpec((D, D // 2), lambda i: (0, 0)),
                  pl.BlockSpec((1, D // 2), lambda i: (0, 0)),
                  pl.BlockSpec((D // 2, 1), lambda i: (0, 0)),
                  pl.BlockSpec((1, 1), lambda i: (0, 0))],
        out_specs=pl.BlockSpec((br, 1), lambda i: (i, 0)),
        out_shape=jax.ShapeDtypeStruct((rows, 1), jnp.float32),
    )(hh, p0, p1, cinv, W0a, W0b, b0.reshape(1, -1), W1, b1.reshape(1, -1),
      W2, b2.reshape(1, 1))


def _bn_scale_shift(st, n_rows, g, b):
    mean = st[0] / n_rows
    var = st[1] / n_rows - mean * mean
    scale = g / jnp.sqrt(var + 1e-5)
    return scale, b - mean * scale


def kernel(h, edge_weight, graph_x, params, edge_index, graph_edge_index,
           graph_batch):
    p = params
    n_nodes = graph_x.shape[0]
    n_h = h.shape[0]
    e2 = graph_edge_index.shape[1]
    e1 = edge_index.shape[1]

    gb = graph_batch.astype(jnp.int32)

    gsrc = graph_edge_index[0].astype(jnp.int32)
    gdst = graph_edge_index[1].astype(jnp.int32)
    order_a = jnp.argsort(gdst)
    src_a = gsrc[order_a]
    dst_a = gdst[order_a]
    nwin = n_nodes // NG
    ptr_a = jnp.pad(
        jnp.searchsorted(dst_a, jnp.arange(0, (nwin + 1) * NG, NG)
                         ).astype(jnp.int32), (0, 32 - (nwin + 1)))

    e1_pad = ((e1 // NW + CH - 1) // CH) * CH * NW
    pad_b = e1_pad - e1
    src_b = jnp.pad(edge_index[0].astype(jnp.int32), (0, pad_b))
    dst_b = jnp.pad(edge_index[1].astype(jnp.int32), (0, pad_b))
    ew_b = jnp.pad(edge_weight, ((0, 0), (0, pad_b)))

    gptr = jnp.searchsorted(gb, jnp.arange(NG + 1)).astype(jnp.int32)
    cnt = (gptr[1:] - gptr[:-1]).astype(jnp.float32)
    cinv = (1.0 / jnp.maximum(cnt, 1.0)).reshape(NG, 1)

    zstripe = jnp.zeros((ZR, D), jnp.float32)

    vnode = jnp.zeros((NG, D), jnp.float32)
    x = graph_x
    for l in range(5):
        xu, xur = _sc_gather_add(vnode, x, gb)
        msg = _sc_msg_sorted(xur, src_a, dst_a, ptr_a, zstripe, n_nodes)
        t1, st1 = _mm_stats(xu, [msg], p['gin_W1'][l], p['gin_b1'][l],
                            None, None, relu_pre=False, br=640)
        sc1, sh1 = _bn_scale_shift(st1, n_nodes, p['gin_bn1_g'][l],
                                   p['gin_bn1_b'][l])
        t2, st2 = _mm_stats(t1, [], p['gin_W2'][l], p['gin_b2'][l],
                            sc1, sh1, relu_pre=True, br=640)
        sc2, sh2 = _bn_scale_shift(st2, n_nodes, p['gin_bng'][l],
                                   p['gin_bnb'][l])
        x = _affine_ew(t2, sc2, sh2, xu, relu=(l < 4), stats=False, br=640)
        if l < 4:
            parts = _sc_pool(x, gb, zstripe)
            vnode = _vnode_mlp(parts[0, :NG], parts[1, :NG], vnode,
                               p['vn_W1'][l], p['vn_b1'][l],
                               p['vn_W2'][l], p['vn_b2'][l], br=400)

    parts5 = _sc_pool(x, gb, zstripe)

    hh = h
    for i in range(2):
        partsB = _sc_msg_weighted(hh, src_b, dst_b, ew_b[i], zstripe, e1)
        t1, st1 = _mm_stats(hh, [partsB[0, :n_h], partsB[1, :n_h]],
                            p['s_W1'][i], p['s_b1'][i],
                            None, None, relu_pre=False, br=400)
        sc1, sh1 = _bn_scale_shift(st1, n_h, p['s_bn1_g'][i], p['s_bn1_b'][i])
        t2, st2 = _mm_stats(t1, [], p['s_W2'][i], p['s_b2'][i],
                            sc1, sh1, relu_pre=True, br=400)
        scA, shA = _bn_scale_shift(st2, n_h, p['s_bnA_g'][i], p['s_bnA_b'][i])
        u3, st3 = _affine_ew(t2, scA, shA, None, relu=True, stats=True, br=400)
        scO, shO = _bn_scale_shift(st3, n_h, p['s_bno_g'][i], p['s_bno_b'][i])
        hh = _affine_ew(u3, scO, shO, None, relu=True, stats=False, br=400)

    y = _readout(hh, parts5[0, :NG], parts5[1, :NG], cinv,
                 p['mr_W0'][:D], p['mr_W0'][D:], p['mr_b0'],
                 p['mr_W1'], p['mr_b1'], p['mr_W2'], p['mr_b2'], br=400)
    return y


# speedup vs baseline: 1.6704x; 1.0498x over previous
import functools

import jax
import jax.numpy as jnp
from jax import lax
from jax.experimental import pallas as pl
from jax.experimental.pallas import tpu as pltpu
from jax.experimental.pallas import tpu_sc as plsc

NW = 32
CH = 128
D = 128
NG = 10000
ACC = 10112
ZR = 632


def _mesh():
    return plsc.VectorSubcoreMesh(core_axis_name="c", subcore_axis_name="s",
                                  num_cores=2)


def _extract(g0, g1, i):
    io = lax.broadcasted_iota(jnp.int32, (16,), 0)
    z = jnp.zeros((16,), jnp.int32)
    a = jnp.sum(jnp.where(io == i, g0, z), axis=0)
    b = jnp.sum(jnp.where(io == (i - 16), g1, z), axis=0)
    return a + b


def _sc_gather_add(table, x, idx):
    n_rows = x.shape[0]
    nch = n_rows // CH
    kmax = (nch + NW - 1) // NW

    nb = 2

    @functools.partial(
        pl.kernel,
        mesh=_mesh(),
        out_type=[jax.ShapeDtypeStruct((n_rows, D), jnp.float32),
                  jax.ShapeDtypeStruct((n_rows, D), jnp.float32)],
        scratch_types=[
            pltpu.VMEM((nb, CH), jnp.int32),
            pltpu.VMEM((nb * CH, D), jnp.float32),
            pltpu.VMEM((nb * CH, D), jnp.float32),
            pltpu.VMEM((nb * CH, D), jnp.float32),
            pltpu.SemaphoreType.DMA,
            pltpu.SemaphoreType.DMA,
            pltpu.SemaphoreType.DMA,
        ],
    )
    def k(tab_h, x_h, idx_h, xu_h, xur_h, idx_v, gbuf, xbuf, rbuf,
          dsem, gsem, osem):
        tid = lax.axis_index("s") * 2 + lax.axis_index("c")

        def body(kk, car):
            cs = [tid + (nb * kk + q) * NW for q in range(nb)]

            def in_pairs(q):
                c = cs[q]
                return [(idx_h.at[pl.ds(c * CH, CH)], idx_v.at[q]),
                        (x_h.at[pl.ds(c * CH, CH)],
                         xbuf.at[pl.ds(q * CH, CH)])]

            for q in range(nb):
                @pl.when(cs[q] < nch)
                def _f(q=q):
                    for s_, d_ in in_pairs(q):
                        pltpu.async_copy(s_, d_, dsem)
            for q in range(nb):
                @pl.when(cs[q] < nch)
                def _w(q=q):
                    for s_, d_ in in_pairs(q):
                        pltpu.make_async_copy(s_, d_, dsem).wait()
            for q in range(nb):
                @pl.when(cs[q] < nch)
                def _g(q=q):
                    pltpu.async_copy(tab_h.at[idx_v.at[q]],
                                     gbuf.at[pl.ds(q * CH, CH)], gsem)
            for q in range(nb):
                @pl.when(cs[q] < nch)
                def _gw(q=q):
                    pltpu.make_async_copy(tab_h.at[idx_v.at[q]],
                                          gbuf.at[pl.ds(q * CH, CH)],
                                          gsem).wait()

            def rloop(r, c2):
                for j in range(D // 16):
                    sl = pl.ds(j * 16, 16)
                    v = xbuf[r, sl] + gbuf[r, sl]
                    xbuf[r, sl] = v
                    rbuf[r, sl] = jnp.maximum(v, 0.0)
                return c2

            lax.fori_loop(0, nb * CH, rloop, 0)

            def out_pairs(q):
                c = cs[q]
                return [(xbuf.at[pl.ds(q * CH, CH)],
                         xu_h.at[pl.ds(c * CH, CH)]),
                        (rbuf.at[pl.ds(q * CH, CH)],
                         xur_h.at[pl.ds(c * CH, CH)])]

            for q in range(nb):
                @pl.when(cs[q] < nch)
                def _o(q=q):
                    for s_, d_ in out_pairs(q):
                        pltpu.async_copy(s_, d_, osem)
            for q in range(nb):
                @pl.when(cs[q] < nch)
                def _ow(q=q):
                    for s_, d_ in out_pairs(q):
                        pltpu.make_async_copy(s_, d_, osem).wait()
            return car

        lax.fori_loop(0, (kmax + nb - 1) // nb, body, 0)

    return k(table, x, idx)


def _sc_msg_sorted(xur, src_s, dst_s, ptr, zeros, n_nodes):
    e = src_s.shape[0]
    nwin = n_nodes // NG

    @functools.partial(
        pl.kernel,
        mesh=_mesh(),
        out_type=jax.ShapeDtypeStruct((n_nodes, D), jnp.float32),
        scratch_types=[
            pltpu.VMEM((32,), jnp.int32),
            pltpu.VMEM((2, CH), jnp.int32),
            pltpu.VMEM((2, CH), jnp.int32),
            pltpu.VMEM((2, CH), jnp.int32),
            pltpu.VMEM((2 * CH, D), jnp.float32),
            pltpu.VMEM_SHARED((ACC, D), jnp.float32),
            pltpu.SemaphoreType.DMA,
            pltpu.SemaphoreType.DMA,
        ],
    )
    def k(tab_h, src_h, dst_h, ptr_h, z_h, out_h,
          ptr_v, idx_v, dst_v, sidx, gbuf, acc, dsem, gsem):
        cid = lax.axis_index("c")
        sid = lax.axis_index("s")
        pltpu.sync_copy(ptr_h, ptr_v)
        io = lax.broadcasted_iota(jnp.int32, (16,), 0)

        for wi in range(nwin // 2):
            w = 2 * wi + cid
            lo = w * NG
            pv = ptr_v[pl.ds(w, 16)]
            e0 = pv[0]
            e1 = pv[1]
            pltpu.sync_copy(z_h, acc.at[pl.ds(sid * ZR, ZR)])
            plsc.subcore_barrier()
            per = (e1 - e0 + 15) // 16
            m0 = e0 + sid * per
            m1 = jnp.minimum(m0 + per, e1)
            k0 = m0 // CH
            k1 = (m1 + CH - 1) // CH

            def cloop(bb, car):
                ks = [k0 + bb * 2 + q for q in range(2)]

                def io_pairs(q):
                    a = ks[q] * CH
                    return [(src_h.at[pl.ds(a, CH)], idx_v.at[q]),
                            (dst_h.at[pl.ds(a, CH)], dst_v.at[q])]

                for q in range(2):
                    @pl.when(ks[q] < k1)
                    def _f(q=q):
                        for s_, d_ in io_pairs(q):
                            pltpu.async_copy(s_, d_, dsem)
                for q in range(2):
                    @pl.when(ks[q] < k1)
                    def _w(q=q):
                        for s_, d_ in io_pairs(q):
                            pltpu.make_async_copy(s_, d_, dsem).wait()
                for q in range(2):
                    @pl.when(ks[q] < k1)
                    def _g(q=q):
                        pltpu.async_copy(tab_h.at[idx_v.at[q]],
                                         gbuf.at[pl.ds(q * CH, CH)], gsem)
                for q in range(2):
                    a16 = ks[q] * CH
                    for j in range(CH // 16):
                        sl = pl.ds(j * 16, 16)
                        eix = io + (a16 + j * 16)
                        valid = (eix >= m0) & (eix < m1)
                        sidx[q, sl] = jnp.where(valid, dst_v[q, sl] - lo,
                                                ACC - 16 + (eix & 7))
                for q in range(2):
                    @pl.when(ks[q] < k1)
                    def _gw(q=q):
                        pltpu.make_async_copy(tab_h.at[idx_v.at[q]],
                                              gbuf.at[pl.ds(q * CH, CH)],
                                              gsem).wait()
                for q in range(2):
                    @pl.when(ks[q] < k1)
                    def _s(q=q):
                        pltpu.sync_copy(gbuf.at[pl.ds(q * CH, CH)],
                                        acc.at[sidx.at[q]], add=True)
                return car

            lax.fori_loop(0, (k1 - k0 + 1) // 2, cloop, 0)
            plsc.subcore_barrier()
            pltpu.sync_copy(acc.at[pl.ds(sid * 624, 624)],
                            out_h.at[pl.ds(lo + sid * 624, 624)])

            @pl.when(sid == 15)
            def _tail():
                pltpu.sync_copy(acc.at[pl.ds(9984, 16)],
                                out_h.at[pl.ds(lo + 9984, 16)])

            plsc.subcore_barrier()

    return k(xur, src_s, dst_s, ptr, zeros)


def _sc_msg_weighted(hh, src_p, dst_p, ew_p, zeros, n_edges_real):
    e_pad = src_p.shape[0]
    per_tile = n_edges_real // NW
    nchk = (per_tile + CH - 1) // CH

    @functools.partial(
        pl.kernel,
        mesh=_mesh(),
        out_type=jax.ShapeDtypeStruct((2, ACC, D), jnp.float32),
        scratch_types=[
            pltpu.VMEM((CH,), jnp.int32),
            pltpu.VMEM((CH,), jnp.int32),
            pltpu.VMEM((CH,), jnp.int32),
            pltpu.VMEM((CH,), jnp.float32),
            pltpu.VMEM((CH, D), jnp.float32),
            pltpu.VMEM_SHARED((ACC, D), jnp.float32),
            pltpu.SemaphoreType.DMA,
        ],
    )
    def k(tab_h, src_h, dst_h, ew_h, z_h, out_h,
          idx_v, dst_v, sidx, wbuf, gbuf, acc, gsem):
        cid = lax.axis_index("c")
        sid = lax.axis_index("s")
        tid = cid * 16 + sid
        base = tid * per_tile
        m1 = base + per_tile
        pltpu.sync_copy(z_h, acc.at[pl.ds(sid * ZR, ZR)])
        plsc.subcore_barrier()
        io = lax.broadcasted_iota(jnp.int32, (16,), 0)

        def cloop(kc, car):
            a = base + kc * CH
            pltpu.sync_copy(src_h.at[pl.ds(a, CH)], idx_v)
            cp = pltpu.async_copy(tab_h.at[idx_v], gbuf, gsem)
            pltpu.sync_copy(dst_h.at[pl.ds(a, CH)], dst_v)
            pltpu.sync_copy(ew_h.at[pl.ds(a, CH)], wbuf)
            for j in range(CH // 16):
                sl = pl.ds(j * 16, 16)
                eix = io + (a + j * 16)
                valid = eix < m1
                sidx[sl] = jnp.where(valid, dst_v[sl], ACC - 16 + (eix & 7))
            cp.wait()

            def gloop(g, car2):
                wg = wbuf[pl.ds(g * 16, 16)]
                for kk in range(16):
                    r = g * 16 + kk
                    wb = lax.gather(
                        wg,
                        jnp.full((16, 1), kk, jnp.int32),
                        lax.GatherDimensionNumbers(
                            offset_dims=(), collapsed_slice_dims=(0,),
                            start_index_map=(0,)),
                        (1,), mode=lax.GatherScatterMode.PROMISE_IN_BOUNDS)
                    for j in range(D // 16):
                        sl = pl.ds(j * 16, 16)
                        gbuf[r, sl] = gbuf[r, sl] * wb
                return car2

            lax.fori_loop(0, CH // 16, gloop, 0)
            pltpu.sync_copy(gbuf, acc.at[sidx], add=True)
            return car

        lax.fori_loop(0, nchk, cloop, 0)
        plsc.subcore_barrier()
        pltpu.sync_copy(acc.at[pl.ds(sid * ZR, ZR)],
                        out_h.at[cid, pl.ds(sid * ZR, ZR)])

    return k(hh, src_p, dst_p, ew_p, zeros)


def _sc_pool(x, gb, zeros):
    n_rows = x.shape[0]
    nch_half = (n_rows // 2) // CH

    @functools.partial(
        pl.kernel,
        mesh=_mesh(),
        out_type=jax.ShapeDtypeStruct((2, ACC, D), jnp.float32),
        scratch_types=[
            pltpu.VMEM((CH,), jnp.int32),
            pltpu.VMEM((CH, D), jnp.float32),
            pltpu.VMEM_SHARED((ACC, D), jnp.float32),
        ],
    )
    def k(x_h, gb_h, z_h, out_h, gb_v, gbuf, acc):
        cid = lax.axis_index("c")
        sid = lax.axis_index("s")
        pltpu.sync_copy(z_h, acc.at[pl.ds(sid * ZR, ZR)])
        plsc.subcore_barrier()
        kmax = (nch_half + 15) // 16

        def cloop(kk, car):
            c = sid + kk * 16

            @pl.when(c < nch_half)
            def _do():
                base = (cid * nch_half + c) * CH
                pltpu.sync_copy(x_h.at[pl.ds(base, CH)], gbuf)
                pltpu.sync_copy(gb_h.at[pl.ds(base, CH)], gb_v)
                pltpu.sync_copy(gbuf, acc.at[gb_v], add=True)
            return car

        lax.fori_loop(0, kmax, cloop, 0)
        plsc.subcore_barrier()
        pltpu.sync_copy(acc.at[pl.ds(sid * ZR, ZR)],
                        out_h.at[cid, pl.ds(sid * ZR, ZR)])

    return k(x, gb, zeros)


def _mm_stats(x, adds, W, b, scale, shift, *, relu_pre, br):
    rows, kdim = x.shape
    ncols = W.shape[1]
    grid = (rows // br,)
    n_add = len(adds)
    has_aff = scale is not None

    def body(*refs):
        it = iter(refs)
        x_ref = next(it)
        add_refs = [next(it) for _ in range(n_add)]
        sc_ref = next(it) if has_aff else None
        sh_ref = next(it) if has_aff else None
        w_ref = next(it)
        b_ref = next(it)
        y_ref = next(it)
        st_ref = next(it)
        xx = x_ref[...]
        for ar in add_refs:
            xx = xx + ar[...]
        if has_aff:
            xx = xx * sc_ref[...] + sh_ref[...]
            if relu_pre:
                xx = jnp.maximum(xx, 0.0)
        y = jnp.dot(xx, w_ref[...], preferred_element_type=jnp.float32)
        y = y + b_ref[...]
        y_ref[...] = y

        @pl.when(pl.program_id(0) == 0)
        def _init():
            st_ref[...] = jnp.zeros_like(st_ref)
            st_ref[2:3, :] = jnp.sum(y, axis=0, keepdims=True) / br

        c = st_ref[2:3, :]
        yc = y - c
        st_ref[0:1, :] += jnp.sum(yc, axis=0, keepdims=True)
        st_ref[1:2, :] += jnp.sum(yc * yc, axis=0, keepdims=True)

    in_specs = [pl.BlockSpec((br, kdim), lambda i: (i, 0))]
    ins = [x]
    for a in adds:
        in_specs.append(pl.BlockSpec((br, kdim), lambda i: (i, 0)))
        ins.append(a)
    if has_aff:
        in_specs += [pl.BlockSpec((1, kdim), lambda i: (0, 0))] * 2
        ins += [scale.reshape(1, kdim), shift.reshape(1, kdim)]
    in_specs += [pl.BlockSpec((kdim, ncols), lambda i: (0, 0)),
                 pl.BlockSpec((1, ncols), lambda i: (0, 0))]
    ins += [W, b.reshape(1, ncols)]

    y, st = pl.pallas_call(
        body,
        grid=grid,
        in_specs=in_specs,
        out_specs=[pl.BlockSpec((br, ncols), lambda i: (i, 0)),
                   pl.BlockSpec((8, ncols), lambda i: (0, 0))],
        out_shape=[jax.ShapeDtypeStruct((rows, ncols), jnp.float32),
                   jax.ShapeDtypeStruct((8, ncols), jnp.float32)],
    )(*ins)
    return y, st


def _affine_ew(z, scale, shift, res, *, relu, stats, br):
    rows, ncols = z.shape
    grid = (rows // br,)
    has_res = res is not None

    def body(*refs):
        it = iter(refs)
        z_ref = next(it)
        sc_ref = next(it)
        sh_ref = next(it)
        r_ref = next(it) if has_res else None
        y_ref = next(it)
        st_ref = next(it) if stats else None
        y = z_ref[...] * sc_ref[...] + sh_ref[...]
        if relu:
            y = jnp.maximum(y, 0.0)
        if has_res:
            y = y + r_ref[...]
        y_ref[...] = y
        if stats:
            @pl.when(pl.program_id(0) == 0)
            def _init():
                st_ref[...] = jnp.zeros_like(st_ref)
                st_ref[2:3, :] = jnp.sum(y, axis=0, keepdims=True) / br
            c = st_ref[2:3, :]
            yc = y - c
            st_ref[0:1, :] += jnp.sum(yc, axis=0, keepdims=True)
            st_ref[1:2, :] += jnp.sum(yc * yc, axis=0, keepdims=True)

    in_specs = [pl.BlockSpec((br, ncols), lambda i: (i, 0)),
                pl.BlockSpec((1, ncols), lambda i: (0, 0)),
                pl.BlockSpec((1, ncols), lambda i: (0, 0))]
    ins = [z, scale.reshape(1, ncols), shift.reshape(1, ncols)]
    if has_res:
        in_specs.append(pl.BlockSpec((br, ncols), lambda i: (i, 0)))
        ins.append(res)
    out_specs = [pl.BlockSpec((br, ncols), lambda i: (i, 0))]
    out_shape = [jax.ShapeDtypeStruct((rows, ncols), jnp.float32)]
    if stats:
        out_specs.append(pl.BlockSpec((8, ncols), lambda i: (0, 0)))
        out_shape.append(jax.ShapeDtypeStruct((8, ncols), jnp.float32))
    out = pl.pallas_call(
        body, grid=grid, in_specs=in_specs, out_specs=out_specs,
        out_shape=out_shape)(*ins)
    return out if stats else out[0]


def _vnode_mlp(p0, p1, vn, W1, b1, W2, b2, *, br):
    rows = vn.shape[0]
    grid = (rows // br,)

    def body(p0_ref, p1_ref, v_ref, w1_ref, b1_ref, w2_ref, b2_ref, o_ref):
        t = jnp.dot(p0_ref[...] + p1_ref[...] + v_ref[...], w1_ref[...],
                    preferred_element_type=jnp.float32) + b1_ref[...]
        t = jnp.maximum(t, 0.0)
        u = jnp.dot(t, w2_ref[...],
                    preferred_element_type=jnp.float32) + b2_ref[...]
        o_ref[...] = v_ref[...] + jnp.maximum(u, 0.0)

    return pl.pallas_call(
        body, grid=grid,
        in_specs=[pl.BlockSpec((br, D), lambda i: (i, 0)),
                  pl.BlockSpec((br, D), lambda i: (i, 0)),
                  pl.BlockSpec((br, D), lambda i: (i, 0)),
                  pl.BlockSpec((D, 2 * D), lambda i: (0, 0)),
                  pl.BlockSpec((1, 2 * D), lambda i: (0, 0)),
                  pl.BlockSpec((2 * D, D), lambda i: (0, 0)),
                  pl.BlockSpec((1, D), lambda i: (0, 0))],
        out_specs=pl.BlockSpec((br, D), lambda i: (i, 0)),
        out_shape=jax.ShapeDtypeStruct((rows, D), jnp.float32),
    )(p0, p1, vn, W1, b1.reshape(1, -1), W2, b2.reshape(1, -1))


def _readout(hh, p0, p1, cinv, W0a, W0b, b0, W1, b1, W2, b2, *, br):
    rows = hh.shape[0]
    grid = (rows // br,)

    def body(h_ref, p0_ref, p1_ref, c_ref, wa_ref, wb_ref, b0_ref,
             w1_ref, b1_ref, w2_ref, b2_ref, o_ref):
        ph = (p0_ref[...] + p1_ref[...]) * c_ref[...]
        y = (jnp.dot(h_ref[...], wa_ref[...], preferred_element_type=jnp.float32)
             + jnp.dot(ph, wb_ref[...], preferred_element_type=jnp.float32)
             + b0_ref[...])
        y = jnp.maximum(y, 0.0)
        y = jnp.dot(y, w1_ref[...], preferred_element_type=jnp.float32) + b1_ref[...]
        y = jnp.maximum(y, 0.0)
        y = jnp.dot(y, w2_ref[...], preferred_element_type=jnp.float32) + b2_ref[...]
        o_ref[...] = y

    return pl.pallas_call(
        body, grid=grid,
        in_specs=[pl.BlockSpec((br, D), lambda i: (i, 0)),
                  pl.BlockSpec((br, D), lambda i: (i, 0)),
                  pl.BlockSpec((br, D), lambda i: (i, 0)),
                  pl.BlockSpec((br, 1), lambda i: (i, 0)),
                  pl.BlockSpec((D, D), lambda i: (0, 0)),
                  pl.BlockSpec((D, D), lambda i: (0, 0)),
                  pl.BlockSpec((1, D), lambda i: (0, 0)),
                  pl.BlockSpec((D, D // 2), lambda i: (0, 0)),
                  pl.BlockSpec((1, D // 2), lambda i: (0, 0)),
                  pl.BlockSpec((D // 2, 1), lambda i: (0, 0)),
                  pl.BlockSpec((1, 1), lambda i: (0, 0))],
        out_specs=pl.BlockSpec((br, 1), lambda i: (i, 0)),
        out_shape=jax.ShapeDtypeStruct((rows, 1), jnp.float32),
    )(hh, p0, p1, cinv, W0a, W0b, b0.reshape(1, -1), W1, b1.reshape(1, -1),
      W2, b2.reshape(1, 1))


def _bn_scale_shift(st, n_rows, g, b):
    dm = st[0] / n_rows
    mean = st[2] + dm
    var = st[1] / n_rows - dm * dm
    scale = g / jnp.sqrt(var + 1e-5)
    return scale, b - mean * scale


def kernel(h, edge_weight, graph_x, params, edge_index, graph_edge_index,
           graph_batch):
    p = params
    n_nodes = graph_x.shape[0]
    n_h = h.shape[0]
    e2 = graph_edge_index.shape[1]
    e1 = edge_index.shape[1]

    gb = graph_batch.astype(jnp.int32)

    gsrc = graph_edge_index[0].astype(jnp.int32)
    gdst = graph_edge_index[1].astype(jnp.int32)
    order_a = jnp.argsort(gdst)
    src_a = gsrc[order_a]
    dst_a = gdst[order_a]
    nwin = n_nodes // NG
    ptr_a = jnp.pad(
        jnp.searchsorted(dst_a, jnp.arange(0, (nwin + 1) * NG, NG)
                         ).astype(jnp.int32), (0, 32 - (nwin + 1)))

    e1_pad = ((e1 // NW + CH - 1) // CH) * CH * NW
    pad_b = e1_pad - e1
    src_b = jnp.pad(edge_index[0].astype(jnp.int32), (0, pad_b))
    dst_b = jnp.pad(edge_index[1].astype(jnp.int32), (0, pad_b))
    ew_b = jnp.pad(edge_weight, ((0, 0), (0, pad_b)))

    gptr = jnp.searchsorted(gb, jnp.arange(NG + 1)).astype(jnp.int32)
    cnt = (gptr[1:] - gptr[:-1]).astype(jnp.float32)
    cinv = (1.0 / jnp.maximum(cnt, 1.0)).reshape(NG, 1)

    zstripe = jnp.zeros((ZR, D), jnp.float32)

    vnode = jnp.zeros((NG, D), jnp.float32)
    x = graph_x
    for l in range(5):
        xu, xur = _sc_gather_add(vnode, x, gb)
        msg = _sc_msg_sorted(xur, src_a, dst_a, ptr_a, zstripe, n_nodes)
        t1, st1 = _mm_stats(xu, [msg], p['gin_W1'][l], p['gin_b1'][l],
                            None, None, relu_pre=False, br=640)
        sc1, sh1 = _bn_scale_shift(st1, n_nodes, p['gin_bn1_g'][l],
                                   p['gin_bn1_b'][l])
        t2, st2 = _mm_stats(t1, [], p['gin_W2'][l], p['gin_b2'][l],
                            sc1, sh1, relu_pre=True, br=640)
        sc2, sh2 = _bn_scale_shift(st2, n_nodes, p['gin_bng'][l],
                                   p['gin_bnb'][l])
        x = _affine_ew(t2, sc2, sh2, xu, relu=(l < 4), stats=False, br=640)
        if l < 4:
            parts = _sc_pool(x, gb, zstripe)
            vnode = _vnode_mlp(parts[0, :NG], parts[1, :NG], vnode,
                               p['vn_W1'][l], p['vn_b1'][l],
                               p['vn_W2'][l], p['vn_b2'][l], br=400)

    parts5 = _sc_pool(x, gb, zstripe)

    hh = h
    for i in range(2):
        partsB = _sc_msg_weighted(hh, src_b, dst_b, ew_b[i], zstripe, e1)
        t1, st1 = _mm_stats(hh, [partsB[0, :n_h], partsB[1, :n_h]],
                            p['s_W1'][i], p['s_b1'][i],
                            None, None, relu_pre=False, br=400)
        sc1, sh1 = _bn_scale_shift(st1, n_h, p['s_bn1_g'][i], p['s_bn1_b'][i])
        t2, st2 = _mm_stats(t1, [], p['s_W2'][i], p['s_b2'][i],
                            sc1, sh1, relu_pre=True, br=400)
        scA, shA = _bn_scale_shift(st2, n_h, p['s_bnA_g'][i], p['s_bnA_b'][i])
        u3, st3 = _affine_ew(t2, scA, shA, None, relu=True, stats=True, br=400)
        scO, shO = _bn_scale_shift(st3, n_h, p['s_bno_g'][i], p['s_bno_b'][i])
        hh = _affine_ew(u3, scO, shO, None, relu=True, stats=False, br=400)

    y = _readout(hh, parts5[0, :NG], parts5[1, :NG], cinv,
                 p['mr_W0'][:D], p['mr_W0'][D:], p['mr_b0'],
                 p['mr_W1'], p['mr_b1'], p['mr_W2'], p['mr_b2'], br=400)
    return y
